# baseline (speedup 1.0000x reference)
import functools

import jax
import jax.numpy as jnp
from jax import lax
from jax.experimental import pallas as pl
from jax.experimental.pallas import tpu as pltpu
from jax.experimental.pallas import tpu_sc as plsc

CHUNK = 64


@functools.lru_cache(maxsize=None)
def _make_sc_kernel(batch, seq_length, embed_dim, dtype):
    info = plsc.get_sparse_core_info()
    num_workers = info.num_cores * info.num_subcores
    rows_per_w = seq_length // num_workers
    n_chunks = rows_per_w // CHUNK

    mesh = plsc.VectorSubcoreMesh(core_axis_name="c", subcore_axis_name="s")

    @functools.partial(
        pl.kernel,
        mesh=mesh,
        out_type=jax.ShapeDtypeStruct((batch, seq_length, embed_dim), dtype),
        scratch_types=[pltpu.VMEM((CHUNK, embed_dim), dtype)],
    )
    def k(emb_hbm, out_hbm, buf):
        wid = lax.axis_index("s") * info.num_cores + lax.axis_index("c")
        base = wid * rows_per_w
        for ci in range(n_chunks):
            r0 = base + ci * CHUNK
            pltpu.sync_copy(emb_hbm.at[pl.ds(r0, CHUNK)], buf)
            for b in range(batch):
                pltpu.sync_copy(buf, out_hbm.at[b, pl.ds(r0, CHUNK)])

    return k


def kernel(input_ids, embedding):
    batch, seq_length = input_ids.shape
    k = _make_sc_kernel(batch, seq_length, embedding.shape[1],
                        embedding.dtype)
    return k(embedding)
